# Initial kernel scaffold; baseline (speedup 1.0000x reference)
import jax
import jax.numpy as jnp
from jax.experimental import pallas as pl


def kernel(q, k, v, Wq, Wk, Wv, Wout):
    raise NotImplementedError("write your pallas kernel here")


# speedup vs baseline: 127.6275x; 127.6275x over previous
import jax
import jax.numpy as jnp
from jax.experimental import pallas as pl

SPAN = 128
STRIDE = 4
LQ = 2048
HEADS = 12
DQK = 64
DV = 64
DIM = 768
KV_USED = (LQ - 1) // STRIDE + 1


def _mha_kernel(q_ref, k_ref, v_ref, wq_ref, wk_ref, wv_ref, wout_ref, out_ref):
    h = pl.program_id(0)
    f32 = jnp.float32
    qh = jax.lax.dot_general(q_ref[...], wq_ref[...],
                             (((1,), (1,)), ((), ())),
                             preferred_element_type=f32)
    kh = jax.lax.dot_general(k_ref[...], wk_ref[...],
                             (((1,), (1,)), ((), ())),
                             preferred_element_type=f32)
    vh = jax.lax.dot_general(v_ref[...], wv_ref[...],
                             (((1,), (1,)), ((), ())),
                             preferred_element_type=f32)
    s = jax.lax.dot_general(qh, kh, (((1,), (1,)), ((), ())),
                            preferred_element_type=f32) * (1.0 / (DQK ** 0.5))
    col = jax.lax.broadcasted_iota(jnp.int32, (LQ, KV_USED), 0)
    j4 = STRIDE * jax.lax.broadcasted_iota(jnp.int32, (LQ, KV_USED), 1)
    valid = (j4 <= col) & (col - j4 < SPAN)
    s = jnp.where(valid, s, -jnp.inf)
    m = jnp.max(s, axis=1, keepdims=True)
    e = jnp.exp(s - m)
    p = e / jnp.sum(e, axis=1, keepdims=True)
    oh = jax.lax.dot_general(p, vh, (((1,), (0,)), ((), ())),
                             preferred_element_type=f32)
    contrib = jax.lax.dot_general(oh, wout_ref[...], (((1,), (0,)), ((), ())),
                                  preferred_element_type=f32)

    @pl.when(h == 0)
    def _():
        out_ref[...] = contrib

    @pl.when(h != 0)
    def _():
        out_ref[...] = out_ref[...] + contrib


def kernel(q, k, v, Wq, Wk, Wv, Wout):
    batch = q.shape[0]
    q2 = q.reshape(batch * LQ, DIM)
    k2 = k.reshape(-1, DIM)
    v2 = v.reshape(-1, DIM)
    out = pl.pallas_call(
        _mha_kernel,
        grid=(HEADS,),
        in_specs=[
            pl.BlockSpec((LQ, DIM), lambda h: (0, 0)),
            pl.BlockSpec((KV_USED, DIM), lambda h: (0, 0)),
            pl.BlockSpec((KV_USED, DIM), lambda h: (0, 0)),
            pl.BlockSpec((DQK, DIM), lambda h: (h, 0)),
            pl.BlockSpec((DQK, DIM), lambda h: (h, 0)),
            pl.BlockSpec((DV, DIM), lambda h: (h, 0)),
            pl.BlockSpec((DV, DIM), lambda h: (h, 0)),
        ],
        out_specs=pl.BlockSpec((LQ, DIM), lambda h: (0, 0)),
        out_shape=jax.ShapeDtypeStruct((LQ, DIM), jnp.float32),
    )(q2, k2, v2, Wq, Wk, Wv, Wout.T)
    return out.reshape(batch, LQ, DIM)


# speedup vs baseline: 239.7101x; 1.8782x over previous
import jax
import jax.numpy as jnp
from jax.experimental import pallas as pl

SPAN = 128
STRIDE = 4
LQ = 2048
HEADS = 12
DQK = 64
DV = 64
DIM = 768
KV_USED = (LQ - 1) // STRIDE + 1
SCALE = 1.0 / (DQK ** 0.5)


def _dot_t(a, b):
    return jax.lax.dot_general(a, b, (((1,), (1,)), ((), ())),
                               preferred_element_type=jnp.float32)


def _mha_kernel(q_ref, k_ref, v_ref, wq_ref, wk_ref, wv_ref, wout_ref, out_ref):
    Qf = _dot_t(q_ref[...], wq_ref[...])
    Kf = _dot_t(k_ref[...], wk_ref[...])
    Vf = _dot_t(v_ref[...], wv_ref[...])
    col = jax.lax.broadcasted_iota(jnp.int32, (LQ, KV_USED), 0)
    j4 = STRIDE * jax.lax.broadcasted_iota(jnp.int32, (LQ, KV_USED), 1)
    valid = (j4 <= col) & (col - j4 < SPAN)
    bias = jnp.where(valid, 0.0, -jnp.inf).astype(jnp.float32)
    ohs = []
    for h in range(HEADS):
        qh = Qf[:, h * DQK:(h + 1) * DQK]
        kh = Kf[:, h * DQK:(h + 1) * DQK]
        vh = Vf[:, h * DV:(h + 1) * DV]
        s = _dot_t(qh, kh) * SCALE + bias
        m = jnp.max(s, axis=1, keepdims=True)
        e = jnp.exp(s - m)
        p = e / jnp.sum(e, axis=1, keepdims=True)
        ohs.append(jax.lax.dot_general(p, vh, (((1,), (0,)), ((), ())),
                                       preferred_element_type=jnp.float32))
    qkv = jnp.concatenate(ohs, axis=1)
    out_ref[...] = _dot_t(qkv, wout_ref[...])


def kernel(q, k, v, Wq, Wk, Wv, Wout):
    batch = q.shape[0]
    q2 = q.reshape(batch * LQ, DIM)
    k2 = k.reshape(-1, DIM)
    v2 = v.reshape(-1, DIM)
    out = pl.pallas_call(
        _mha_kernel,
        grid=(1,),
        in_specs=[
            pl.BlockSpec((LQ, DIM), lambda i: (0, 0)),
            pl.BlockSpec((KV_USED, DIM), lambda i: (0, 0)),
            pl.BlockSpec((KV_USED, DIM), lambda i: (0, 0)),
            pl.BlockSpec((HEADS * DQK, DIM), lambda i: (0, 0)),
            pl.BlockSpec((HEADS * DQK, DIM), lambda i: (0, 0)),
            pl.BlockSpec((HEADS * DV, DIM), lambda i: (0, 0)),
            pl.BlockSpec((DIM, HEADS * DV), lambda i: (0, 0)),
        ],
        out_specs=pl.BlockSpec((LQ, DIM), lambda i: (0, 0)),
        out_shape=jax.ShapeDtypeStruct((LQ, DIM), jnp.float32),
    )(q2, k2, v2, Wq, Wk, Wv, Wout)
    return out.reshape(batch, LQ, DIM)
